# baseline (speedup 1.0000x reference)
import functools
import math

import jax
import jax.numpy as jnp
from jax import lax
from jax.experimental import pallas as pl
from jax.experimental.pallas import tpu as pltpu
from jax.experimental.pallas import tpu_sc as plsc

D_MODEL = 64
SCALE = math.sqrt(D_MODEL)

_NC = 2
_NS = 16
_NW = _NC * _NS
_LANES = 16

_B = 4096 * 200
_B_PER_W = _B // _NW
_CH = 512
_N_CHUNKS = _B_PER_W // _CH


def _emb_body(idx_hbm, lut_hbm, out_hbm, idx_v, rows_v, sem):
    wid = lax.axis_index("s") * _NC + lax.axis_index("c")
    base = wid * _B_PER_W

    def chunk_body(i, carry):
        off = base + i * _CH
        pltpu.sync_copy(idx_hbm.at[pl.ds(off, _CH)], idx_v)
        pltpu.async_copy(lut_hbm.at[idx_v], rows_v, sem).wait()

        def row_body(r, c):
            for j in range(D_MODEL // _LANES):
                s = pl.ds(j * _LANES, _LANES)
                rows_v[r, s] = rows_v[r, s] * SCALE
            return c

        lax.fori_loop(0, _CH, row_body, 0, unroll=4)
        pltpu.sync_copy(rows_v, out_hbm.at[pl.ds(off, _CH)])
        return carry

    lax.fori_loop(0, _N_CHUNKS, chunk_body, 0)


@jax.jit
def _emb(idx_flat, lut):
    mesh = plsc.VectorSubcoreMesh(core_axis_name="c", subcore_axis_name="s")
    fn = pl.kernel(
        _emb_body,
        out_type=jax.ShapeDtypeStruct((_B, D_MODEL), jnp.float32),
        mesh=mesh,
        scratch_types=[
            pltpu.VMEM((_CH,), jnp.int32),
            pltpu.VMEM((_CH, D_MODEL), jnp.float32),
            pltpu.SemaphoreType.DMA,
        ],
        compiler_params=pltpu.CompilerParams(use_tc_tiling_on_sc=False),
    )
    return fn(idx_flat, lut)


def kernel(x, lut):
    out = _emb(x.reshape(-1), lut)
    return out.reshape(x.shape[0], x.shape[1], D_MODEL)


# speedup vs baseline: 1.0897x; 1.0897x over previous
import math

import jax
import jax.numpy as jnp
from jax import lax
from jax.experimental import pallas as pl
from jax.experimental.pallas import tpu as pltpu
from jax.experimental.pallas import tpu_sc as plsc

D_MODEL = 64
SCALE = math.sqrt(D_MODEL)

_NC = 2
_NS = 16
_NW = _NC * _NS
_LANES = 16

_B = 4096 * 200
_B_PER_W = _B // _NW
_CH = 512
_N_CHUNKS = _B_PER_W // _CH


def _emb_body(idx_hbm, lut_hbm, out_hbm,
              idx_v, rows0, rows1, gsem0, gsem1, osem0, osem1):
    wid = lax.axis_index("s") * _NC + lax.axis_index("c")
    base = wid * _B_PER_W
    pltpu.sync_copy(idx_hbm.at[pl.ds(base, _B_PER_W)], idx_v)

    rows = (rows0, rows1)
    gsem = (gsem0, gsem1)
    osem = (osem0, osem1)

    def g_start(ci, b):
        pltpu.async_copy(
            lut_hbm.at[idx_v.at[pl.ds(ci * _CH, _CH)]], rows[b], gsem[b])

    def g_wait(b):
        pltpu.make_async_copy(
            lut_hbm.at[idx_v.at[pl.ds(0, _CH)]], rows[b], gsem[b]).wait()

    def o_start(ci, b):
        pltpu.async_copy(
            rows[b], out_hbm.at[pl.ds(base + ci * _CH, _CH)], osem[b])

    def o_wait(b):
        pltpu.make_async_copy(
            rows[b], out_hbm.at[pl.ds(base, _CH)], osem[b]).wait()

    def scale(b):
        r = rows[b]

        def row_body(rr, c):
            for j in range(D_MODEL // _LANES):
                s = pl.ds(j * _LANES, _LANES)
                r[rr, s] = r[rr, s] * SCALE
            return c

        lax.fori_loop(0, _CH, row_body, 0, unroll=4)

    g_start(0, 0)
    g_wait(0)
    g_start(1, 1)
    scale(0)
    o_start(0, 0)

    def pair_body(p, carry):
        ci = 2 * p + 1
        g_wait(1)
        o_wait(0)
        g_start(ci + 1, 0)
        scale(1)
        o_start(ci, 1)

        g_wait(0)
        o_wait(1)
        g_start(ci + 2, 1)
        scale(0)
        o_start(ci + 1, 0)
        return carry

    lax.fori_loop(0, (_N_CHUNKS - 2) // 2, pair_body, 0)

    g_wait(1)
    scale(1)
    o_start(_N_CHUNKS - 1, 1)
    o_wait(0)
    o_wait(1)


@jax.jit
def _emb(idx_flat, lut):
    mesh = plsc.VectorSubcoreMesh(core_axis_name="c", subcore_axis_name="s")
    fn = pl.kernel(
        _emb_body,
        out_type=jax.ShapeDtypeStruct((_B, D_MODEL), jnp.float32),
        mesh=mesh,
        scratch_types=[
            pltpu.VMEM((_B_PER_W,), jnp.int32),
            pltpu.VMEM((_CH, D_MODEL), jnp.float32),
            pltpu.VMEM((_CH, D_MODEL), jnp.float32),
            pltpu.SemaphoreType.DMA,
            pltpu.SemaphoreType.DMA,
            pltpu.SemaphoreType.DMA,
            pltpu.SemaphoreType.DMA,
        ],
        compiler_params=pltpu.CompilerParams(use_tc_tiling_on_sc=False),
    )
    return fn(idx_flat, lut)


def kernel(x, lut):
    out = _emb(x.reshape(-1), lut)
    return out.reshape(x.shape[0], x.shape[1], D_MODEL)
